# baseline (speedup 1.0000x reference)
import functools

import jax
import jax.numpy as jnp
from jax import lax
from jax.experimental import pallas as pl
from jax.experimental.pallas import tpu as pltpu
from jax.experimental.pallas import tpu_sc as plsc

N = 10000
E = 320000
D = 128
H = 8
DK = 16
HPC = H // 2
DH = HPC * DK

NP = 10240
NT = 16
EPT = 20224
EP = NT * EPT
EB = 128
NBLK = EPT // EB
RPT = NP // NT
R = 1280
WZ = 80


def _dot(a, b):
    return jnp.dot(a, b, preferred_element_type=jnp.float32)


def _ln(x, g, b):
    m = jnp.mean(x, axis=1, keepdims=True)
    xc = x - m
    v = jnp.mean(xc * xc, axis=1, keepdims=True)
    return xc * lax.rsqrt(v + 1e-5) * g + b


def _write_qkv(y, q_ref, kv_ref):
    q_ref[0] = y[:, :DH]
    q_ref[1] = y[:, DH:D]
    kv_ref[0] = y[:, D:D + 2 * DH]
    kv_ref[1] = y[:, D + 2 * DH:]


def _stage_a_body(x_ref, wh_ref, bh_ref, wp_ref, bp_ref, h_ref, q_ref,
                  kv_ref):
    h = _dot(x_ref[...], wh_ref[...]) + bh_ref[...]
    h_ref[...] = h
    _write_qkv(_dot(h, wp_ref[...]) + bp_ref[...], q_ref, kv_ref)


def _attn_ffn(wvz_ref, h_ref, wo_ref, bo_ref, g1_ref, be1_ref, w1_ref, b1_ref,
              w2_ref, b2_ref, g2_ref, be2_ref):
    wvz = wvz_ref[...]
    pieces = []
    for hh in range(H):
        c, j = hh // HPC, hh % HPC
        zc = wvz[c, :, DH + j:DH + j + 1] + 1e-6
        pieces.append(wvz[c, :, j * DK:(j + 1) * DK] / zc)
    attn = jnp.concatenate(pieces, axis=1)
    h1 = h_ref[...] + _dot(attn, wo_ref[...]) + bo_ref[...]
    h1 = _ln(h1, g1_ref[...], be1_ref[...])
    ff = jnp.maximum(_dot(h1, w1_ref[...]) + b1_ref[...], 0.0)
    ff = _dot(ff, w2_ref[...]) + b2_ref[...]
    return _ln(h1 + ff, g2_ref[...], be2_ref[...])


def _stage_mid_body(wvz_ref, h_ref, wo_ref, bo_ref, g1_ref, be1_ref, w1_ref,
                    b1_ref, w2_ref, b2_ref, g2_ref, be2_ref, wp_ref, bp_ref,
                    h_out, q_out, kv_out):
    h2 = _attn_ffn(wvz_ref, h_ref, wo_ref, bo_ref, g1_ref, be1_ref, w1_ref,
                   b1_ref, w2_ref, b2_ref, g2_ref, be2_ref)
    h_out[...] = h2
    _write_qkv(_dot(h2, wp_ref[...]) + bp_ref[...], q_out, kv_out)


def _stage_fin_body(wvz_ref, h_ref, wo_ref, bo_ref, g1_ref, be1_ref, w1_ref,
                    b1_ref, w2_ref, b2_ref, g2_ref, be2_ref, wout_ref,
                    bout_ref, o_ref):
    h2 = _attn_ffn(wvz_ref, h_ref, wo_ref, bo_ref, g1_ref, be1_ref, w1_ref,
                   b1_ref, w2_ref, b2_ref, g2_ref, be2_ref)
    o_ref[...] = _dot(h2, wout_ref[...]) + bout_ref[...]


def _row_spec():
    return pl.BlockSpec((R, D), lambda i: (i, 0))


def _w_spec(r, c):
    return pl.BlockSpec((r, c), lambda i: (0, 0))


_GRID = NP // R

_QKV_OUT_SPECS = [
    pl.BlockSpec((2, R, DH), lambda i: (0, i, 0)),
    pl.BlockSpec((2, R, 2 * DH), lambda i: (0, i, 0)),
]
_QKV_OUT_SHAPE = [
    jax.ShapeDtypeStruct((2, NP, DH), jnp.float32),
    jax.ShapeDtypeStruct((2, NP, 2 * DH), jnp.float32),
]

_MID_W_SPECS = [_w_spec(D, D), _w_spec(1, D), _w_spec(1, D), _w_spec(1, D),
                _w_spec(D, 2 * D), _w_spec(1, 2 * D), _w_spec(2 * D, D),
                _w_spec(1, D), _w_spec(1, D), _w_spec(1, D)]

_STAGE_A_KW = dict(
    grid=(_GRID,),
    in_specs=[_row_spec(), _w_spec(D, D), _w_spec(1, D), _w_spec(D, 3 * D),
              _w_spec(1, 3 * D)],
    out_specs=[_row_spec()] + _QKV_OUT_SPECS,
    out_shape=[jax.ShapeDtypeStruct((NP, D), jnp.float32)] + _QKV_OUT_SHAPE,
)

_STAGE_MID_KW = dict(
    grid=(_GRID,),
    in_specs=[pl.BlockSpec((2, R, WZ), lambda i: (0, i, 0)), _row_spec()] +
             _MID_W_SPECS + [_w_spec(D, 3 * D), _w_spec(1, 3 * D)],
    out_specs=[_row_spec()] + _QKV_OUT_SPECS,
    out_shape=[jax.ShapeDtypeStruct((NP, D), jnp.float32)] + _QKV_OUT_SHAPE,
)

_STAGE_FIN_KW = dict(
    grid=(_GRID,),
    in_specs=[pl.BlockSpec((2, R, WZ), lambda i: (0, i, 0)), _row_spec()] +
             _MID_W_SPECS + [_w_spec(D, D), _w_spec(1, D)],
    out_specs=[_row_spec()],
    out_shape=[jax.ShapeDtypeStruct((NP, D), jnp.float32)],
)

_stage_a = pl.pallas_call(_stage_a_body, **_STAGE_A_KW)
_stage_mid = pl.pallas_call(_stage_mid_body, **_STAGE_MID_KW)
_stage_fin = pl.pallas_call(_stage_fin_body, **_STAGE_FIN_KW)


def _sc_edge_body(kv_hbm, q_hbm, ei_hbm, out_hbm,
                  is0, is1, id0, id1, id2, kv0, kv1, qb0, qb1, wz0, wz1,
                  accum, gs0, gs1, ss0, ss1):
    c = lax.axis_index("c")
    s = lax.axis_index("s")
    idx_s = (is0, is1)
    idx_d = (id0, id1, id2)
    kvb = (kv0, kv1)
    qb = (qb0, qb1)
    wzb = (wz0, wz1)
    gsem = (gs0, gs1)
    ssem = (ss0, ss1)

    def _zrow(i, _):
        for j in range(WZ // 16):
            wz0[i, pl.ds(j * 16, 16)] = jnp.zeros((16,), jnp.float32)
        return 0
    lax.fori_loop(0, EB, _zrow, 0)
    for k in range(RPT // EB):
        pltpu.sync_copy(wz0, accum.at[pl.ds(s * RPT + k * EB, EB)])
    plsc.subcore_barrier()

    lane = lax.iota(jnp.int32, 16)
    perms = [lane ^ sh for sh in (8, 4, 2, 1)]
    masks = [lane == hh for hh in range(HPC)]
    ebase = s * EPT
    kvc = kv_hbm.at[c]
    qc = q_hbm.at[c]

    def _issue(g, d, b):
        base = ebase + b * EB
        pltpu.sync_copy(ei_hbm.at[0, pl.ds(base, EB)], idx_s[g])
        pltpu.sync_copy(ei_hbm.at[1, pl.ds(base, EB)], idx_d[d])
        pltpu.async_copy(kvc.at[idx_s[g]], kvb[g], gsem[g])
        pltpu.async_copy(qc.at[idx_d[d]], qb[g], gsem[g])

    def _wait_gather(g, d):
        pltpu.make_async_copy(kvc.at[idx_s[g]], kvb[g], gsem[g]).wait()
        pltpu.make_async_copy(qc.at[idx_d[d]], qb[g], gsem[g]).wait()

    def _wait_scatter(w, d):
        pltpu.make_async_copy(wzb[w], accum.at[idx_d[d]], ssem[w]).wait()

    def _compute(t):
        kvbuf, qbuf, wvzbuf = kvb[t], qb[t], wzb[t]

        def _edge(i, _):
            zvec = jnp.zeros((16,), jnp.float32)
            for hh in range(HPC):
                kvec = kvbuf[i, pl.ds(hh * DK, DK)]
                qvec = qbuf[i, pl.ds(hh * DK, DK)]
                p = kvec * qvec
                for perm in perms:
                    p = p + p.at[perm].get(mode="promise_in_bounds")
                evec = jnp.exp(jnp.clip(p, -5.0, 5.0))
                vvec = kvbuf[i, pl.ds(DH + hh * DK, DK)]
                wvzbuf[i, pl.ds(hh * DK, DK)] = evec * vvec
                zvec = jnp.where(masks[hh], evec, zvec)
            wvzbuf[i, pl.ds(DH, DK)] = zvec
            return 0
        lax.fori_loop(0, EB, _edge, 0, unroll=4)

    def _step(b, bm, issue_next=True, wait_sc=True):
        g, gn, d, dn = bm % 2, (bm + 1) % 2, bm % 3, (bm + 1) % 3
        if wait_sc:
            _wait_scatter(g, dn)
        if issue_next:
            _issue(gn, dn, b + 1)
        _wait_gather(g, d)
        _compute(g)
        pltpu.async_copy(wzb[g], accum.at[idx_d[d]], ssem[g], add=True)

    _issue(0, 0, 0)
    _step(0, 0, wait_sc=False)
    _step(1, 1, wait_sc=False)

    def _sixpack(i, _):
        b = 6 * i + 2
        for k in range(6):
            _step(b + k, (2 + k) % 6)
        return 0
    lax.fori_loop(0, (NBLK - 8) // 6, _sixpack, 0)

    for k in range(6):
        b = NBLK - 6 + k
        _step(b, b % 6, issue_next=(k < 5))
    _wait_scatter((NBLK - 2) % 2, (NBLK - 2) % 3)
    _wait_scatter((NBLK - 1) % 2, (NBLK - 1) % 3)

    plsc.subcore_barrier()
    pltpu.sync_copy(accum.at[pl.ds(s * RPT, RPT)],
                    out_hbm.at[c, pl.ds(s * RPT, RPT)])


@functools.lru_cache(maxsize=None)
def _get_sc_edge():
    return pl.kernel(
        _sc_edge_body,
        out_type=jax.ShapeDtypeStruct((2, NP, WZ), jnp.float32),
        mesh=plsc.VectorSubcoreMesh(core_axis_name="c", subcore_axis_name="s"),
        compiler_params=pltpu.CompilerParams(use_tc_tiling_on_sc=False),
        scratch_types=(
            [pltpu.VMEM((EB,), jnp.int32)] * 5 +
            [pltpu.VMEM((EB, 2 * DH), jnp.float32)] * 2 +
            [pltpu.VMEM((EB, DH), jnp.float32)] * 2 +
            [pltpu.VMEM((EB, WZ), jnp.float32)] * 2 +
            [pltpu.VMEM_SHARED((NP, WZ), jnp.float32)] +
            [pltpu.SemaphoreType.DMA] * 4
        ),
    )


def _sc_edge(kv, q, eip):
    return _get_sc_edge()(kv, q, eip)


def _proj_w(p):
    scale = 1.0 / (DK ** 0.5)
    w = jnp.concatenate([p["Wq"] * scale, p["Wk"][:, :DH], p["Wv"][:, :DH],
                         p["Wk"][:, DH:], p["Wv"][:, DH:]], axis=1)
    b = jnp.concatenate([p["bq"] * scale, p["bk"][:DH], p["bv"][:DH],
                         p["bk"][DH:], p["bv"][DH:]])
    return w, b.reshape(1, 3 * D)


def _mid_w(p):
    return [p["Wo"], p["bo"].reshape(1, D), p["g1"].reshape(1, D),
            p["be1"].reshape(1, D), p["W1"], p["b1"].reshape(1, 2 * D),
            p["W2"], p["b2"].reshape(1, D), p["g2"].reshape(1, D),
            p["be2"].reshape(1, D)]


def kernel(x, params, edge_index):
    xp = jnp.zeros((NP, D), jnp.float32).at[:N].set(x)
    eip = jnp.pad(edge_index, ((0, 0), (0, EP - E)), constant_values=N)

    l0, l1 = params["layers"]
    wp0, bp0 = _proj_w(l0)
    wp1, bp1 = _proj_w(l1)

    h0, q0, kv0 = _stage_a(xp, params["W_h"], params["b_h"].reshape(1, D),
                           wp0, bp0)
    wvz0 = _sc_edge(kv0, q0, eip)
    h1, q1, kv1 = _stage_mid(wvz0, h0, *_mid_w(l0), wp1, bp1)
    wvz1 = _sc_edge(kv1, q1, eip)
    (out,) = _stage_fin(wvz1, h1, *_mid_w(l1), params["W_out"],
                        params["b_out"].reshape(1, D))
    return out[:N]


# speedup vs baseline: 2.2327x; 2.2327x over previous
import functools

import jax
import jax.numpy as jnp
from jax import lax
from jax.experimental import pallas as pl
from jax.experimental.pallas import tpu as pltpu
from jax.experimental.pallas import tpu_sc as plsc

N = 10000
E = 320000
D = 128
H = 8
DK = 16
HPC = H // 2
DH = HPC * DK

NP = 10240
NT = 16
EPT = 20224
EP = NT * EPT
EB = 128
NBLK = EPT // EB
RPT = NP // NT
R = 1280
WZ = 80


def _dot(a, b):
    return jnp.dot(a, b, preferred_element_type=jnp.float32)


def _ln(x, g, b):
    m = jnp.mean(x, axis=1, keepdims=True)
    xc = x - m
    v = jnp.mean(xc * xc, axis=1, keepdims=True)
    return xc * lax.rsqrt(v + 1e-5) * g + b


def _write_qkv(y, q_ref, kv_ref):
    q_ref[0] = y[:, :DH]
    q_ref[1] = y[:, DH:D]
    kv_ref[0] = y[:, D:D + 2 * DH]
    kv_ref[1] = y[:, D + 2 * DH:]


def _stage_a_body(x_ref, wh_ref, bh_ref, wp_ref, bp_ref, h_ref, q_ref,
                  kv_ref):
    h = _dot(x_ref[...], wh_ref[...]) + bh_ref[...]
    h_ref[...] = h
    _write_qkv(_dot(h, wp_ref[...]) + bp_ref[...], q_ref, kv_ref)


def _attn_ffn(wvz_ref, h_ref, wo_ref, bo_ref, g1_ref, be1_ref, w1_ref, b1_ref,
              w2_ref, b2_ref, g2_ref, be2_ref):
    wvz = wvz_ref[...]
    pieces = []
    for hh in range(H):
        c, j = hh // HPC, hh % HPC
        zc = wvz[c, :, DH + j:DH + j + 1] + 1e-6
        pieces.append(wvz[c, :, j * DK:(j + 1) * DK] / zc)
    attn = jnp.concatenate(pieces, axis=1)
    h1 = h_ref[...] + _dot(attn, wo_ref[...]) + bo_ref[...]
    h1 = _ln(h1, g1_ref[...], be1_ref[...])
    ff = jnp.maximum(_dot(h1, w1_ref[...]) + b1_ref[...], 0.0)
    ff = _dot(ff, w2_ref[...]) + b2_ref[...]
    return _ln(h1 + ff, g2_ref[...], be2_ref[...])


def _stage_mid_body(wvz_ref, h_ref, wo_ref, bo_ref, g1_ref, be1_ref, w1_ref,
                    b1_ref, w2_ref, b2_ref, g2_ref, be2_ref, wp_ref, bp_ref,
                    h_out, q_out, kv_out):
    h2 = _attn_ffn(wvz_ref, h_ref, wo_ref, bo_ref, g1_ref, be1_ref, w1_ref,
                   b1_ref, w2_ref, b2_ref, g2_ref, be2_ref)
    h_out[...] = h2
    _write_qkv(_dot(h2, wp_ref[...]) + bp_ref[...], q_out, kv_out)


def _stage_fin_body(wvz_ref, h_ref, wo_ref, bo_ref, g1_ref, be1_ref, w1_ref,
                    b1_ref, w2_ref, b2_ref, g2_ref, be2_ref, wout_ref,
                    bout_ref, o_ref):
    h2 = _attn_ffn(wvz_ref, h_ref, wo_ref, bo_ref, g1_ref, be1_ref, w1_ref,
                   b1_ref, w2_ref, b2_ref, g2_ref, be2_ref)
    o_ref[...] = _dot(h2, wout_ref[...]) + bout_ref[...]


def _row_spec():
    return pl.BlockSpec((R, D), lambda i: (i, 0))


def _w_spec(r, c):
    return pl.BlockSpec((r, c), lambda i: (0, 0))


_GRID = NP // R

_QKV_OUT_SPECS = [
    pl.BlockSpec((2, R, DH), lambda i: (0, i, 0)),
    pl.BlockSpec((2, R, 2 * DH), lambda i: (0, i, 0)),
]
_QKV_OUT_SHAPE = [
    jax.ShapeDtypeStruct((2, NP, DH), jnp.float32),
    jax.ShapeDtypeStruct((2, NP, 2 * DH), jnp.float32),
]

_MID_W_SPECS = [_w_spec(D, D), _w_spec(1, D), _w_spec(1, D), _w_spec(1, D),
                _w_spec(D, 2 * D), _w_spec(1, 2 * D), _w_spec(2 * D, D),
                _w_spec(1, D), _w_spec(1, D), _w_spec(1, D)]

_STAGE_A_KW = dict(
    grid=(_GRID,),
    in_specs=[_row_spec(), _w_spec(D, D), _w_spec(1, D), _w_spec(D, 3 * D),
              _w_spec(1, 3 * D)],
    out_specs=[_row_spec()] + _QKV_OUT_SPECS,
    out_shape=[jax.ShapeDtypeStruct((NP, D), jnp.float32)] + _QKV_OUT_SHAPE,
)

_STAGE_MID_KW = dict(
    grid=(_GRID,),
    in_specs=[pl.BlockSpec((2, R, WZ), lambda i: (0, i, 0)), _row_spec()] +
             _MID_W_SPECS + [_w_spec(D, 3 * D), _w_spec(1, 3 * D)],
    out_specs=[_row_spec()] + _QKV_OUT_SPECS,
    out_shape=[jax.ShapeDtypeStruct((NP, D), jnp.float32)] + _QKV_OUT_SHAPE,
)

_STAGE_FIN_KW = dict(
    grid=(_GRID,),
    in_specs=[pl.BlockSpec((2, R, WZ), lambda i: (0, i, 0)), _row_spec()] +
             _MID_W_SPECS + [_w_spec(D, D), _w_spec(1, D)],
    out_specs=[_row_spec()],
    out_shape=[jax.ShapeDtypeStruct((NP, D), jnp.float32)],
)

_stage_a = pl.pallas_call(_stage_a_body, **_STAGE_A_KW)
_stage_mid = pl.pallas_call(_stage_mid_body, **_STAGE_MID_KW)
_stage_fin = pl.pallas_call(_stage_fin_body, **_STAGE_FIN_KW)


def _sc_edge_body(kv_hbm, q_hbm, ei_hbm, out_hbm,
                  is0, is1, id0, id1, id2, kv0, kv1, qb0, qb1, wz0, wz1,
                  accum, gs0, gs1, ss0, ss1):
    c = lax.axis_index("c")
    s = lax.axis_index("s")
    idx_s = (is0, is1)
    idx_d = (id0, id1, id2)
    kvb = (kv0, kv1)
    qb = (qb0, qb1)
    wzb = (wz0, wz1)
    gsem = (gs0, gs1)
    ssem = (ss0, ss1)

    def _zrow(i, _):
        for j in range(WZ // 16):
            wz0[i, pl.ds(j * 16, 16)] = jnp.zeros((16,), jnp.float32)
        return 0
    lax.fori_loop(0, EB, _zrow, 0)
    for k in range(RPT // EB):
        pltpu.sync_copy(wz0, accum.at[pl.ds(s * RPT + k * EB, EB)])
    plsc.subcore_barrier()

    lane = lax.iota(jnp.int32, 16)
    perms = [lane ^ sh for sh in (8, 4, 2, 1)]
    masks = [lane == hh for hh in range(HPC)]
    ebase = s * EPT
    kvc = kv_hbm.at[c]
    qc = q_hbm.at[c]

    def _issue(g, d, b):
        base = ebase + b * EB
        pltpu.sync_copy(ei_hbm.at[0, pl.ds(base, EB)], idx_s[g])
        pltpu.sync_copy(ei_hbm.at[1, pl.ds(base, EB)], idx_d[d])
        pltpu.async_copy(kvc.at[idx_s[g]], kvb[g], gsem[g])
        pltpu.async_copy(qc.at[idx_d[d]], qb[g], gsem[g])

    def _wait_gather(g, d):
        pltpu.make_async_copy(kvc.at[idx_s[g]], kvb[g], gsem[g]).wait()
        pltpu.make_async_copy(qc.at[idx_d[d]], qb[g], gsem[g]).wait()

    def _wait_scatter(w, d):
        pltpu.make_async_copy(wzb[w], accum.at[idx_d[d]], ssem[w]).wait()

    zero16 = lane * 0
    idx4 = lane & 3
    gmasks = [(lane >> 2) == j for j in range(4)]

    def _compute(t):
        kvbuf, qbuf, wvzbuf = kvb[t], qb[t], wzb[t]

        def _quad(g, _):
            i0 = g * 4
            svecs = []
            for j in range(4):
                i = i0 + j
                svec = jnp.zeros((16,), jnp.float32)
                for hh in range(HPC):
                    kvec = kvbuf[i, pl.ds(hh * DK, DK)]
                    qvec = qbuf[i, pl.ds(hh * DK, DK)]
                    p = kvec * qvec
                    for perm in perms:
                        p = p + p.at[perm].get(mode="promise_in_bounds")
                    svec = jnp.where(masks[hh], p, svec)
                svecs.append(svec)
            packed = svecs[0].at[idx4].get(mode="promise_in_bounds")
            for j in range(1, 4):
                rep = svecs[j].at[idx4].get(mode="promise_in_bounds")
                packed = jnp.where(gmasks[j], rep, packed)
            packed = jnp.exp(jnp.clip(packed, -5.0, 5.0))
            for j in range(4):
                i = i0 + j
                evec = packed.at[idx4 + 4 * j].get(mode="promise_in_bounds")
                wvzbuf[i, pl.ds(DH, DK)] = evec
                for hh in range(HPC):
                    eb = evec.at[zero16 + hh].get(mode="promise_in_bounds")
                    vvec = kvbuf[i, pl.ds(DH + hh * DK, DK)]
                    wvzbuf[i, pl.ds(hh * DK, DK)] = eb * vvec
            return 0
        lax.fori_loop(0, EB // 4, _quad, 0)

    def _step(b, bm, issue_next=True, wait_sc=True):
        g, gn, d, dn = bm % 2, (bm + 1) % 2, bm % 3, (bm + 1) % 3
        if wait_sc:
            _wait_scatter(g, dn)
        if issue_next:
            _issue(gn, dn, b + 1)
        _wait_gather(g, d)
        _compute(g)
        pltpu.async_copy(wzb[g], accum.at[idx_d[d]], ssem[g], add=True)

    _issue(0, 0, 0)
    _step(0, 0, wait_sc=False)
    _step(1, 1, wait_sc=False)

    def _sixpack(i, _):
        b = 6 * i + 2
        for k in range(6):
            _step(b + k, (2 + k) % 6)
        return 0
    lax.fori_loop(0, (NBLK - 8) // 6, _sixpack, 0)

    for k in range(6):
        b = NBLK - 6 + k
        _step(b, b % 6, issue_next=(k < 5))
    _wait_scatter((NBLK - 2) % 2, (NBLK - 2) % 3)
    _wait_scatter((NBLK - 1) % 2, (NBLK - 1) % 3)

    plsc.subcore_barrier()
    pltpu.sync_copy(accum.at[pl.ds(s * RPT, RPT)],
                    out_hbm.at[c, pl.ds(s * RPT, RPT)])


@functools.lru_cache(maxsize=None)
def _get_sc_edge():
    return pl.kernel(
        _sc_edge_body,
        out_type=jax.ShapeDtypeStruct((2, NP, WZ), jnp.float32),
        mesh=plsc.VectorSubcoreMesh(core_axis_name="c", subcore_axis_name="s"),
        compiler_params=pltpu.CompilerParams(use_tc_tiling_on_sc=False),
        scratch_types=(
            [pltpu.VMEM((EB,), jnp.int32)] * 5 +
            [pltpu.VMEM((EB, 2 * DH), jnp.float32)] * 2 +
            [pltpu.VMEM((EB, DH), jnp.float32)] * 2 +
            [pltpu.VMEM((EB, WZ), jnp.float32)] * 2 +
            [pltpu.VMEM_SHARED((NP, WZ), jnp.float32)] +
            [pltpu.SemaphoreType.DMA] * 4
        ),
    )


def _sc_edge(kv, q, eip):
    return _get_sc_edge()(kv, q, eip)


def _proj_w(p):
    scale = 1.0 / (DK ** 0.5)
    w = jnp.concatenate([p["Wq"] * scale, p["Wk"][:, :DH], p["Wv"][:, :DH],
                         p["Wk"][:, DH:], p["Wv"][:, DH:]], axis=1)
    b = jnp.concatenate([p["bq"] * scale, p["bk"][:DH], p["bv"][:DH],
                         p["bk"][DH:], p["bv"][DH:]])
    return w, b.reshape(1, 3 * D)


def _mid_w(p):
    return [p["Wo"], p["bo"].reshape(1, D), p["g1"].reshape(1, D),
            p["be1"].reshape(1, D), p["W1"], p["b1"].reshape(1, 2 * D),
            p["W2"], p["b2"].reshape(1, D), p["g2"].reshape(1, D),
            p["be2"].reshape(1, D)]


def kernel(x, params, edge_index):
    xp = jnp.zeros((NP, D), jnp.float32).at[:N].set(x)
    eip = jnp.pad(edge_index, ((0, 0), (0, EP - E)), constant_values=N)

    l0, l1 = params["layers"]
    wp0, bp0 = _proj_w(l0)
    wp1, bp1 = _proj_w(l1)

    h0, q0, kv0 = _stage_a(xp, params["W_h"], params["b_h"].reshape(1, D),
                           wp0, bp0)
    wvz0 = _sc_edge(kv0, q0, eip)
    h1, q1, kv1 = _stage_mid(wvz0, h0, *_mid_w(l0), wp1, bp1)
    wvz1 = _sc_edge(kv1, q1, eip)
    (out,) = _stage_fin(wvz1, h1, *_mid_w(l1), params["W_out"],
                        params["b_out"].reshape(1, D))
    return out[:N]


# speedup vs baseline: 2.2339x; 1.0006x over previous
import functools

import jax
import jax.numpy as jnp
from jax import lax
from jax.experimental import pallas as pl
from jax.experimental.pallas import tpu as pltpu
from jax.experimental.pallas import tpu_sc as plsc

N = 10000
E = 320000
D = 128
H = 8
DK = 16
HPC = H // 2
DH = HPC * DK

NP = 10240
NT = 16
EPT = 20224
EP = NT * EPT
EB = 128
NBLK = EPT // EB
RPT = NP // NT
R = 1280
WZ = 80


def _dot(a, b):
    return jnp.dot(a, b, preferred_element_type=jnp.float32)


def _ln(x, g, b):
    m = jnp.mean(x, axis=1, keepdims=True)
    xc = x - m
    v = jnp.mean(xc * xc, axis=1, keepdims=True)
    return xc * lax.rsqrt(v + 1e-5) * g + b


def _write_qkv(y, q_ref, kv_ref):
    q_ref[0] = y[:, :DH]
    q_ref[1] = y[:, DH:D]
    kv_ref[0] = y[:, D:D + 2 * DH]
    kv_ref[1] = y[:, D + 2 * DH:]


def _stage_a_body(x_ref, wh_ref, bh_ref, wp_ref, bp_ref, h_ref, q_ref,
                  kv_ref):
    h = _dot(x_ref[...], wh_ref[...]) + bh_ref[...]
    h_ref[...] = h
    _write_qkv(_dot(h, wp_ref[...]) + bp_ref[...], q_ref, kv_ref)


def _attn_ffn(wvz_ref, h_ref, wo_ref, bo_ref, g1_ref, be1_ref, w1_ref, b1_ref,
              w2_ref, b2_ref, g2_ref, be2_ref):
    wvz = wvz_ref[...]
    pieces = []
    for hh in range(H):
        c, j = hh // HPC, hh % HPC
        zc = wvz[c, :, DH + j:DH + j + 1] + 1e-6
        pieces.append(wvz[c, :, j * DK:(j + 1) * DK] / zc)
    attn = jnp.concatenate(pieces, axis=1)
    h1 = h_ref[...] + _dot(attn, wo_ref[...]) + bo_ref[...]
    h1 = _ln(h1, g1_ref[...], be1_ref[...])
    ff = jnp.maximum(_dot(h1, w1_ref[...]) + b1_ref[...], 0.0)
    ff = _dot(ff, w2_ref[...]) + b2_ref[...]
    return _ln(h1 + ff, g2_ref[...], be2_ref[...])


def _stage_mid_body(wvz_ref, h_ref, wo_ref, bo_ref, g1_ref, be1_ref, w1_ref,
                    b1_ref, w2_ref, b2_ref, g2_ref, be2_ref, wp_ref, bp_ref,
                    h_out, q_out, kv_out):
    h2 = _attn_ffn(wvz_ref, h_ref, wo_ref, bo_ref, g1_ref, be1_ref, w1_ref,
                   b1_ref, w2_ref, b2_ref, g2_ref, be2_ref)
    h_out[...] = h2
    _write_qkv(_dot(h2, wp_ref[...]) + bp_ref[...], q_out, kv_out)


def _stage_fin_body(wvz_ref, h_ref, wo_ref, bo_ref, g1_ref, be1_ref, w1_ref,
                    b1_ref, w2_ref, b2_ref, g2_ref, be2_ref, wout_ref,
                    bout_ref, o_ref):
    h2 = _attn_ffn(wvz_ref, h_ref, wo_ref, bo_ref, g1_ref, be1_ref, w1_ref,
                   b1_ref, w2_ref, b2_ref, g2_ref, be2_ref)
    o_ref[...] = _dot(h2, wout_ref[...]) + bout_ref[...]


def _row_spec():
    return pl.BlockSpec((R, D), lambda i: (i, 0))


def _w_spec(r, c):
    return pl.BlockSpec((r, c), lambda i: (0, 0))


_GRID = NP // R

_QKV_OUT_SPECS = [
    pl.BlockSpec((2, R, DH), lambda i: (0, i, 0)),
    pl.BlockSpec((2, R, 2 * DH), lambda i: (0, i, 0)),
]
_QKV_OUT_SHAPE = [
    jax.ShapeDtypeStruct((2, NP, DH), jnp.float32),
    jax.ShapeDtypeStruct((2, NP, 2 * DH), jnp.float32),
]

_MID_W_SPECS = [_w_spec(D, D), _w_spec(1, D), _w_spec(1, D), _w_spec(1, D),
                _w_spec(D, 2 * D), _w_spec(1, 2 * D), _w_spec(2 * D, D),
                _w_spec(1, D), _w_spec(1, D), _w_spec(1, D)]

_STAGE_A_KW = dict(
    grid=(_GRID,),
    in_specs=[_row_spec(), _w_spec(D, D), _w_spec(1, D), _w_spec(D, 3 * D),
              _w_spec(1, 3 * D)],
    out_specs=[_row_spec()] + _QKV_OUT_SPECS,
    out_shape=[jax.ShapeDtypeStruct((NP, D), jnp.float32)] + _QKV_OUT_SHAPE,
)

_STAGE_MID_KW = dict(
    grid=(_GRID,),
    in_specs=[pl.BlockSpec((2, R, WZ), lambda i: (0, i, 0)), _row_spec()] +
             _MID_W_SPECS + [_w_spec(D, 3 * D), _w_spec(1, 3 * D)],
    out_specs=[_row_spec()] + _QKV_OUT_SPECS,
    out_shape=[jax.ShapeDtypeStruct((NP, D), jnp.float32)] + _QKV_OUT_SHAPE,
)

_STAGE_FIN_KW = dict(
    grid=(_GRID,),
    in_specs=[pl.BlockSpec((2, R, WZ), lambda i: (0, i, 0)), _row_spec()] +
             _MID_W_SPECS + [_w_spec(D, D), _w_spec(1, D)],
    out_specs=[_row_spec()],
    out_shape=[jax.ShapeDtypeStruct((NP, D), jnp.float32)],
)

_stage_a = pl.pallas_call(_stage_a_body, **_STAGE_A_KW)
_stage_mid = pl.pallas_call(_stage_mid_body, **_STAGE_MID_KW)
_stage_fin = pl.pallas_call(_stage_fin_body, **_STAGE_FIN_KW)


def _sc_edge_body(kv_hbm, q_hbm, ei_hbm, out_hbm,
                  is0, is1, id0, id1, id2, kv0, kv1, qb0, qb1, wz0, wz1,
                  accum, gs0, gs1, ss0, ss1):
    c = lax.axis_index("c")
    s = lax.axis_index("s")
    idx_s = (is0, is1)
    idx_d = (id0, id1, id2)
    kvb = (kv0, kv1)
    qb = (qb0, qb1)
    wzb = (wz0, wz1)
    gsem = (gs0, gs1)
    ssem = (ss0, ss1)

    def _zrow(i, _):
        for j in range(WZ // 16):
            wz0[i, pl.ds(j * 16, 16)] = jnp.zeros((16,), jnp.float32)
        return 0
    lax.fori_loop(0, EB, _zrow, 0)
    for k in range(RPT // EB):
        pltpu.sync_copy(wz0, accum.at[pl.ds(s * RPT + k * EB, EB)])
    plsc.subcore_barrier()

    lane = lax.iota(jnp.int32, 16)
    perms = [lane ^ sh for sh in (8, 4, 2, 1)]
    masks = [lane == hh for hh in range(HPC)]
    ebase = s * EPT
    kvc = kv_hbm.at[c]
    qc = q_hbm.at[c]

    def _issue(g, d, b):
        base = ebase + b * EB
        pltpu.sync_copy(ei_hbm.at[0, pl.ds(base, EB)], idx_s[g])
        pltpu.sync_copy(ei_hbm.at[1, pl.ds(base, EB)], idx_d[d])
        pltpu.async_copy(kvc.at[idx_s[g]], kvb[g], gsem[g])
        pltpu.async_copy(qc.at[idx_d[d]], qb[g], gsem[g])

    def _wait_gather(g, d):
        pltpu.make_async_copy(kvc.at[idx_s[g]], kvb[g], gsem[g]).wait()
        pltpu.make_async_copy(qc.at[idx_d[d]], qb[g], gsem[g]).wait()

    def _wait_scatter(w, d):
        pltpu.make_async_copy(wzb[w], accum.at[idx_d[d]], ssem[w]).wait()

    zero16 = lane * 0
    idx4 = lane & 3
    gmasks = [(lane >> 2) == j for j in range(4)]

    def _compute(t):
        kvbuf, qbuf, wvzbuf = kvb[t], qb[t], wzb[t]

        def _quad(g, _):
            i0 = g * 4
            svecs = []
            for j in range(4):
                i = i0 + j
                svec = jnp.zeros((16,), jnp.float32)
                for hh in range(HPC):
                    kvec = kvbuf[i, pl.ds(hh * DK, DK)]
                    qvec = qbuf[i, pl.ds(hh * DK, DK)]
                    p = kvec * qvec
                    for perm in perms:
                        p = p + p.at[perm].get(mode="promise_in_bounds")
                    svec = jnp.where(masks[hh], p, svec)
                svecs.append(svec)
            packed = svecs[0].at[idx4].get(mode="promise_in_bounds")
            for j in range(1, 4):
                rep = svecs[j].at[idx4].get(mode="promise_in_bounds")
                packed = jnp.where(gmasks[j], rep, packed)
            packed = jnp.exp(jnp.clip(packed, -5.0, 5.0))
            for j in range(4):
                i = i0 + j
                evec = packed.at[idx4 + 4 * j].get(mode="promise_in_bounds")
                wvzbuf[i, pl.ds(DH, DK)] = evec
                for hh in range(HPC):
                    eb = evec.at[zero16 + hh].get(mode="promise_in_bounds")
                    vvec = kvbuf[i, pl.ds(DH + hh * DK, DK)]
                    wvzbuf[i, pl.ds(hh * DK, DK)] = eb * vvec
            return 0
        lax.fori_loop(0, EB // 4, _quad, 0, unroll=2)

    def _step(b, bm, issue_next=True, wait_sc=True):
        g, gn, d, dn = bm % 2, (bm + 1) % 2, bm % 3, (bm + 1) % 3
        if wait_sc:
            _wait_scatter(g, dn)
        if issue_next:
            _issue(gn, dn, b + 1)
        _wait_gather(g, d)
        _compute(g)
        pltpu.async_copy(wzb[g], accum.at[idx_d[d]], ssem[g], add=True)

    _issue(0, 0, 0)
    _step(0, 0, wait_sc=False)
    _step(1, 1, wait_sc=False)

    def _sixpack(i, _):
        b = 6 * i + 2
        for k in range(6):
            _step(b + k, (2 + k) % 6)
        return 0
    lax.fori_loop(0, (NBLK - 8) // 6, _sixpack, 0)

    for k in range(6):
        b = NBLK - 6 + k
        _step(b, b % 6, issue_next=(k < 5))
    _wait_scatter((NBLK - 2) % 2, (NBLK - 2) % 3)
    _wait_scatter((NBLK - 1) % 2, (NBLK - 1) % 3)

    plsc.subcore_barrier()
    pltpu.sync_copy(accum.at[pl.ds(s * RPT, RPT)],
                    out_hbm.at[c, pl.ds(s * RPT, RPT)])


@functools.lru_cache(maxsize=None)
def _get_sc_edge():
    return pl.kernel(
        _sc_edge_body,
        out_type=jax.ShapeDtypeStruct((2, NP, WZ), jnp.float32),
        mesh=plsc.VectorSubcoreMesh(core_axis_name="c", subcore_axis_name="s"),
        compiler_params=pltpu.CompilerParams(use_tc_tiling_on_sc=False),
        scratch_types=(
            [pltpu.VMEM((EB,), jnp.int32)] * 5 +
            [pltpu.VMEM((EB, 2 * DH), jnp.float32)] * 2 +
            [pltpu.VMEM((EB, DH), jnp.float32)] * 2 +
            [pltpu.VMEM((EB, WZ), jnp.float32)] * 2 +
            [pltpu.VMEM_SHARED((NP, WZ), jnp.float32)] +
            [pltpu.SemaphoreType.DMA] * 4
        ),
    )


def _sc_edge(kv, q, eip):
    return _get_sc_edge()(kv, q, eip)


def _proj_w(p):
    scale = 1.0 / (DK ** 0.5)
    w = jnp.concatenate([p["Wq"] * scale, p["Wk"][:, :DH], p["Wv"][:, :DH],
                         p["Wk"][:, DH:], p["Wv"][:, DH:]], axis=1)
    b = jnp.concatenate([p["bq"] * scale, p["bk"][:DH], p["bv"][:DH],
                         p["bk"][DH:], p["bv"][DH:]])
    return w, b.reshape(1, 3 * D)


def _mid_w(p):
    return [p["Wo"], p["bo"].reshape(1, D), p["g1"].reshape(1, D),
            p["be1"].reshape(1, D), p["W1"], p["b1"].reshape(1, 2 * D),
            p["W2"], p["b2"].reshape(1, D), p["g2"].reshape(1, D),
            p["be2"].reshape(1, D)]


def kernel(x, params, edge_index):
    xp = jnp.zeros((NP, D), jnp.float32).at[:N].set(x)
    eip = jnp.pad(edge_index, ((0, 0), (0, EP - E)), constant_values=N)

    l0, l1 = params["layers"]
    wp0, bp0 = _proj_w(l0)
    wp1, bp1 = _proj_w(l1)

    h0, q0, kv0 = _stage_a(xp, params["W_h"], params["b_h"].reshape(1, D),
                           wp0, bp0)
    wvz0 = _sc_edge(kv0, q0, eip)
    h1, q1, kv1 = _stage_mid(wvz0, h0, *_mid_w(l0), wp1, bp1)
    wvz1 = _sc_edge(kv1, q1, eip)
    (out,) = _stage_fin(wvz1, h1, *_mid_w(l1), params["W_out"],
                        params["b_out"].reshape(1, D))
    return out[:N]


# speedup vs baseline: 2.2909x; 1.0255x over previous
import functools

import jax
import jax.numpy as jnp
from jax import lax
from jax.experimental import pallas as pl
from jax.experimental.pallas import tpu as pltpu
from jax.experimental.pallas import tpu_sc as plsc

N = 10000
E = 320000
D = 128
H = 8
DK = 16
HPC = H // 2
DH = HPC * DK

NP = 10240
NT = 16
EPT = 20224
EP = NT * EPT
EB = 128
NBLK = EPT // EB
RPT = NP // NT
R = 1280
WZ = 80


def _dot(a, b):
    return jnp.dot(a, b, preferred_element_type=jnp.float32)


def _ln(x, g, b):
    m = jnp.mean(x, axis=1, keepdims=True)
    xc = x - m
    v = jnp.mean(xc * xc, axis=1, keepdims=True)
    return xc * lax.rsqrt(v + 1e-5) * g + b


def _write_qkv(y, q_ref, kv_ref):
    q_ref[0] = y[:, :DH]
    q_ref[1] = y[:, DH:D]
    kv_ref[0] = y[:, D:D + 2 * DH]
    kv_ref[1] = y[:, D + 2 * DH:]


def _stage_a_body(x_ref, wh_ref, bh_ref, wp_ref, bp_ref, h_ref, q_ref,
                  kv_ref):
    h = _dot(x_ref[...], wh_ref[...]) + bh_ref[...]
    h_ref[...] = h
    _write_qkv(_dot(h, wp_ref[...]) + bp_ref[...], q_ref, kv_ref)


def _attn_ffn(wvz_ref, h_ref, wo_ref, bo_ref, g1_ref, be1_ref, w1_ref, b1_ref,
              w2_ref, b2_ref, g2_ref, be2_ref):
    wvz = wvz_ref[...]
    pieces = []
    for hh in range(H):
        c, j = hh // HPC, hh % HPC
        zc = wvz[c, :, DH + j:DH + j + 1] + 1e-6
        pieces.append(wvz[c, :, j * DK:(j + 1) * DK] / zc)
    attn = jnp.concatenate(pieces, axis=1)
    h1 = h_ref[...] + _dot(attn, wo_ref[...]) + bo_ref[...]
    h1 = _ln(h1, g1_ref[...], be1_ref[...])
    ff = jnp.maximum(_dot(h1, w1_ref[...]) + b1_ref[...], 0.0)
    ff = _dot(ff, w2_ref[...]) + b2_ref[...]
    return _ln(h1 + ff, g2_ref[...], be2_ref[...])


def _stage_mid_body(wvz_ref, h_ref, wo_ref, bo_ref, g1_ref, be1_ref, w1_ref,
                    b1_ref, w2_ref, b2_ref, g2_ref, be2_ref, wp_ref, bp_ref,
                    h_out, q_out, kv_out):
    h2 = _attn_ffn(wvz_ref, h_ref, wo_ref, bo_ref, g1_ref, be1_ref, w1_ref,
                   b1_ref, w2_ref, b2_ref, g2_ref, be2_ref)
    h_out[...] = h2
    _write_qkv(_dot(h2, wp_ref[...]) + bp_ref[...], q_out, kv_out)


def _stage_fin_body(wvz_ref, h_ref, wo_ref, bo_ref, g1_ref, be1_ref, w1_ref,
                    b1_ref, w2_ref, b2_ref, g2_ref, be2_ref, wout_ref,
                    bout_ref, o_ref):
    h2 = _attn_ffn(wvz_ref, h_ref, wo_ref, bo_ref, g1_ref, be1_ref, w1_ref,
                   b1_ref, w2_ref, b2_ref, g2_ref, be2_ref)
    o_ref[...] = _dot(h2, wout_ref[...]) + bout_ref[...]


def _row_spec():
    return pl.BlockSpec((R, D), lambda i: (i, 0))


def _w_spec(r, c):
    return pl.BlockSpec((r, c), lambda i: (0, 0))


_GRID = NP // R

_QKV_OUT_SPECS = [
    pl.BlockSpec((2, R, DH), lambda i: (0, i, 0)),
    pl.BlockSpec((2, R, 2 * DH), lambda i: (0, i, 0)),
]
_QKV_OUT_SHAPE = [
    jax.ShapeDtypeStruct((2, NP, DH), jnp.float32),
    jax.ShapeDtypeStruct((2, NP, 2 * DH), jnp.float32),
]

_MID_W_SPECS = [_w_spec(D, D), _w_spec(1, D), _w_spec(1, D), _w_spec(1, D),
                _w_spec(D, 2 * D), _w_spec(1, 2 * D), _w_spec(2 * D, D),
                _w_spec(1, D), _w_spec(1, D), _w_spec(1, D)]

_STAGE_A_KW = dict(
    grid=(_GRID,),
    in_specs=[_row_spec(), _w_spec(D, D), _w_spec(1, D), _w_spec(D, 3 * D),
              _w_spec(1, 3 * D)],
    out_specs=[_row_spec()] + _QKV_OUT_SPECS,
    out_shape=[jax.ShapeDtypeStruct((NP, D), jnp.float32)] + _QKV_OUT_SHAPE,
)

_STAGE_MID_KW = dict(
    grid=(_GRID,),
    in_specs=[pl.BlockSpec((2, R, WZ), lambda i: (0, i, 0)), _row_spec()] +
             _MID_W_SPECS + [_w_spec(D, 3 * D), _w_spec(1, 3 * D)],
    out_specs=[_row_spec()] + _QKV_OUT_SPECS,
    out_shape=[jax.ShapeDtypeStruct((NP, D), jnp.float32)] + _QKV_OUT_SHAPE,
)

_STAGE_FIN_KW = dict(
    grid=(_GRID,),
    in_specs=[pl.BlockSpec((2, R, WZ), lambda i: (0, i, 0)), _row_spec()] +
             _MID_W_SPECS + [_w_spec(D, D), _w_spec(1, D)],
    out_specs=[_row_spec()],
    out_shape=[jax.ShapeDtypeStruct((NP, D), jnp.float32)],
)

_stage_a = pl.pallas_call(_stage_a_body, **_STAGE_A_KW)
_stage_mid = pl.pallas_call(_stage_mid_body, **_STAGE_MID_KW)
_stage_fin = pl.pallas_call(_stage_fin_body, **_STAGE_FIN_KW)


def _sc_edge_body(kv_hbm, q_hbm, ei_hbm, out_hbm,
                  is0, is1, id0, id1, id2, kv0, kv1, qb0, qb1, wz0, wz1,
                  accum, gs0, gs1, ss0, ss1):
    c = lax.axis_index("c")
    s = lax.axis_index("s")
    idx_s = (is0, is1)
    idx_d = (id0, id1, id2)
    kvb = (kv0, kv1)
    qb = (qb0, qb1)
    wzb = (wz0, wz1)
    gsem = (gs0, gs1)
    ssem = (ss0, ss1)

    def _zrow(i, _):
        for j in range(WZ // 16):
            wz0[i, pl.ds(j * 16, 16)] = jnp.zeros((16,), jnp.float32)
        return 0
    lax.fori_loop(0, EB, _zrow, 0)
    for k in range(RPT // EB):
        pltpu.sync_copy(wz0, accum.at[pl.ds(s * RPT + k * EB, EB)])
    plsc.subcore_barrier()

    lane = lax.iota(jnp.int32, 16)
    perms = [lane ^ sh for sh in (8, 4, 2, 1)]
    masks = [lane == hh for hh in range(HPC)]
    ebase = s * EPT
    kvc = kv_hbm.at[c]
    qc = q_hbm.at[c]

    def _issue(g, d, b):
        base = ebase + b * EB
        pltpu.sync_copy(ei_hbm.at[0, pl.ds(base, EB)], idx_s[g])
        pltpu.sync_copy(ei_hbm.at[1, pl.ds(base, EB)], idx_d[d])
        pltpu.async_copy(kvc.at[idx_s[g]], kvb[g], gsem[g])
        pltpu.async_copy(qc.at[idx_d[d]], qb[g], gsem[g])

    def _wait_gather(g, d):
        pltpu.make_async_copy(kvc.at[idx_s[g]], kvb[g], gsem[g]).wait()
        pltpu.make_async_copy(qc.at[idx_d[d]], qb[g], gsem[g]).wait()

    def _wait_scatter(w, d):
        pltpu.make_async_copy(wzb[w], accum.at[idx_d[d]], ssem[w]).wait()

    zero16 = lane * 0
    idx4 = lane & 3
    gmasks = [(lane >> 2) == j for j in range(4)]
    lo8 = lane < 8
    lo2 = idx4 < 2
    ib8 = (lane & 1) * 8

    def _compute(t):
        kvbuf, qbuf, wvzbuf = kvb[t], qb[t], wzb[t]

        def _quad(g, _):
            i0 = g * 4
            packed = None
            for j in range(4):
                i = i0 + j
                ms = []
                for pr in range(2):
                    halves = []
                    for hh in (2 * pr, 2 * pr + 1):
                        kvec = kvbuf[i, pl.ds(hh * DK, DK)]
                        qvec = qbuf[i, pl.ds(hh * DK, DK)]
                        p = kvec * qvec
                        halves.append(
                            p + p.at[perms[0]].get(mode="promise_in_bounds"))
                    m = jnp.where(lo8, halves[0], halves[1])
                    for perm in perms[1:]:
                        m = m + m.at[perm].get(mode="promise_in_bounds")
                    ms.append(m)
                rep = jnp.where(lo2,
                                ms[0].at[ib8].get(mode="promise_in_bounds"),
                                ms[1].at[ib8].get(mode="promise_in_bounds"))
                packed = rep if j == 0 else jnp.where(gmasks[j], rep, packed)
            packed = jnp.exp(jnp.clip(packed, -5.0, 5.0))
            for j in range(4):
                i = i0 + j
                evec = packed.at[idx4 + 4 * j].get(mode="promise_in_bounds")
                wvzbuf[i, pl.ds(DH, DK)] = evec
                for hh in range(HPC):
                    eb = evec.at[zero16 + hh].get(mode="promise_in_bounds")
                    vvec = kvbuf[i, pl.ds(DH + hh * DK, DK)]
                    wvzbuf[i, pl.ds(hh * DK, DK)] = eb * vvec
            return 0
        lax.fori_loop(0, EB // 4, _quad, 0, unroll=2)

    def _step(b, bm, issue_next=True, wait_sc=True):
        g, gn, d, dn = bm % 2, (bm + 1) % 2, bm % 3, (bm + 1) % 3
        if wait_sc:
            _wait_scatter(g, dn)
        if issue_next:
            _issue(gn, dn, b + 1)
        _wait_gather(g, d)
        _compute(g)
        pltpu.async_copy(wzb[g], accum.at[idx_d[d]], ssem[g], add=True)

    _issue(0, 0, 0)
    _step(0, 0, wait_sc=False)
    _step(1, 1, wait_sc=False)

    def _sixpack(i, _):
        b = 6 * i + 2
        for k in range(6):
            _step(b + k, (2 + k) % 6)
        return 0
    lax.fori_loop(0, (NBLK - 8) // 6, _sixpack, 0)

    for k in range(6):
        b = NBLK - 6 + k
        _step(b, b % 6, issue_next=(k < 5))
    _wait_scatter((NBLK - 2) % 2, (NBLK - 2) % 3)
    _wait_scatter((NBLK - 1) % 2, (NBLK - 1) % 3)

    plsc.subcore_barrier()
    pltpu.sync_copy(accum.at[pl.ds(s * RPT, RPT)],
                    out_hbm.at[c, pl.ds(s * RPT, RPT)])


@functools.lru_cache(maxsize=None)
def _get_sc_edge():
    return pl.kernel(
        _sc_edge_body,
        out_type=jax.ShapeDtypeStruct((2, NP, WZ), jnp.float32),
        mesh=plsc.VectorSubcoreMesh(core_axis_name="c", subcore_axis_name="s"),
        compiler_params=pltpu.CompilerParams(use_tc_tiling_on_sc=False),
        scratch_types=(
            [pltpu.VMEM((EB,), jnp.int32)] * 5 +
            [pltpu.VMEM((EB, 2 * DH), jnp.float32)] * 2 +
            [pltpu.VMEM((EB, DH), jnp.float32)] * 2 +
            [pltpu.VMEM((EB, WZ), jnp.float32)] * 2 +
            [pltpu.VMEM_SHARED((NP, WZ), jnp.float32)] +
            [pltpu.SemaphoreType.DMA] * 4
        ),
    )


def _sc_edge(kv, q, eip):
    return _get_sc_edge()(kv, q, eip)


def _proj_w(p):
    scale = 1.0 / (DK ** 0.5)
    w = jnp.concatenate([p["Wq"] * scale, p["Wk"][:, :DH], p["Wv"][:, :DH],
                         p["Wk"][:, DH:], p["Wv"][:, DH:]], axis=1)
    b = jnp.concatenate([p["bq"] * scale, p["bk"][:DH], p["bv"][:DH],
                         p["bk"][DH:], p["bv"][DH:]])
    return w, b.reshape(1, 3 * D)


def _mid_w(p):
    return [p["Wo"], p["bo"].reshape(1, D), p["g1"].reshape(1, D),
            p["be1"].reshape(1, D), p["W1"], p["b1"].reshape(1, 2 * D),
            p["W2"], p["b2"].reshape(1, D), p["g2"].reshape(1, D),
            p["be2"].reshape(1, D)]


def kernel(x, params, edge_index):
    xp = jnp.zeros((NP, D), jnp.float32).at[:N].set(x)
    eip = jnp.pad(edge_index, ((0, 0), (0, EP - E)), constant_values=N)

    l0, l1 = params["layers"]
    wp0, bp0 = _proj_w(l0)
    wp1, bp1 = _proj_w(l1)

    h0, q0, kv0 = _stage_a(xp, params["W_h"], params["b_h"].reshape(1, D),
                           wp0, bp0)
    wvz0 = _sc_edge(kv0, q0, eip)
    h1, q1, kv1 = _stage_mid(wvz0, h0, *_mid_w(l0), wp1, bp1)
    wvz1 = _sc_edge(kv1, q1, eip)
    (out,) = _stage_fin(wvz1, h1, *_mid_w(l1), params["W_out"],
                        params["b_out"].reshape(1, D))
    return out[:N]
